# Initial kernel scaffold; baseline (speedup 1.0000x reference)
import jax
import jax.numpy as jnp
from jax.experimental import pallas as pl


def kernel(stru_feats, visu_feats, ling_feats, Wl1, bl1, Wl2, bl2, gl, betal, Wv1, bv1, Wv2, bv2, gv, betav):
    raise NotImplementedError("write your pallas kernel here")


# speedup vs baseline: 33.0780x; 33.0780x over previous
import functools

import jax
import jax.numpy as jnp
from jax import lax
from jax.experimental import pallas as pl
from jax.experimental.pallas import tpu as pltpu
from jax.experimental.pallas import tpu_sc as plsc

N = 4096
D = 256
INV_TAU = 2.0
K = 6
IDXW = 128
BR = 256
NB = N // BR
NW = 32
RPW = N // NW
GR = 16
NG = RPW // GR
_PREC = lax.Precision.HIGHEST


def _norm_body(x_ref, o_ref):
    x = x_ref[...]
    nrm = jnp.sqrt(jnp.sum(x * x, axis=1, keepdims=True))
    o_ref[...] = x / (nrm + 1e-12)


def _norm(x):
    return pl.pallas_call(
        _norm_body,
        out_shape=jax.ShapeDtypeStruct((N, D), jnp.float32),
    )(x)


def _fusion_body(x_ref, w1_ref, b1_ref, w2_ref, b2_ref, g_ref, beta_ref,
                 out_ref, nout_ref):
    h = jnp.dot(x_ref[...], w1_ref[...], precision=_PREC,
                preferred_element_type=jnp.float32) + b1_ref[...]
    h = jnp.where(h >= 0, h, 0.01 * h)
    h = jnp.dot(h, w2_ref[...], precision=_PREC,
                preferred_element_type=jnp.float32) + b2_ref[...]
    mean = jnp.mean(h, axis=0, keepdims=True)
    ctr = h - mean
    var = jnp.mean(ctr * ctr, axis=0, keepdims=True)
    h = ctr / jnp.sqrt(var + 1e-5) * g_ref[...] + beta_ref[...]
    out_ref[...] = h
    nrm = jnp.sqrt(jnp.sum(h * h, axis=1, keepdims=True))
    nout_ref[...] = h / (nrm + 1e-12)


def _fusion(x, w1, b1, w2, b2, g, beta):
    return pl.pallas_call(
        _fusion_body,
        out_shape=[jax.ShapeDtypeStruct((N, D), jnp.float32),
                   jax.ShapeDtypeStruct((N, D), jnp.float32)],
    )(x, w1, b1.reshape(1, D), w2, b2.reshape(1, D),
      g.reshape(1, D), beta.reshape(1, D))


def _topk_cols(p):
    colio = lax.broadcasted_iota(jnp.int32, p.shape, 1)
    cur = p
    cols = []
    for _ in range(K):
        m = jnp.max(cur, axis=1, keepdims=True)
        idx = jnp.min(jnp.where(cur == m, colio, N), axis=1, keepdims=True)
        cols.append(idx)
        cur = jnp.where(colio == idx, -jnp.inf, cur)
    cols.append(jnp.zeros((p.shape[0], IDXW - K), jnp.int32))
    return jnp.concatenate(cols, axis=1)


def _sim_body(sn_ref, mnb_ref, mn_ref, sm_ref, rs_ref, stop_ref, mtop_ref):
    mn = mn_ref[...]
    p = lax.dot_general(sn_ref[...], mn, (((1,), (1,)), ((), ())),
                        precision=_PREC, preferred_element_type=jnp.float32)
    smb = jnp.exp(p * INV_TAU)
    sm_ref[...] = smb
    rs_ref[...] = jnp.broadcast_to(
        jnp.sum(smb, axis=1, keepdims=True), (BR, IDXW))
    stop_ref[...] = _topk_cols(p)
    q = lax.dot_general(mnb_ref[...], mn, (((1,), (1,)), ((), ())),
                        precision=_PREC, preferred_element_type=jnp.float32)
    mtop_ref[...] = _topk_cols(q)


def _sim(sn, mn):
    return pl.pallas_call(
        _sim_body,
        grid=(NB,),
        in_specs=[pl.BlockSpec((BR, D), lambda i: (i, 0)),
                  pl.BlockSpec((BR, D), lambda i: (i, 0)),
                  pl.BlockSpec((N, D), lambda i: (0, 0))],
        out_specs=[pl.BlockSpec((BR, N), lambda i: (i, 0)),
                   pl.BlockSpec((BR, IDXW), lambda i: (i, 0)),
                   pl.BlockSpec((BR, IDXW), lambda i: (i, 0)),
                   pl.BlockSpec((BR, IDXW), lambda i: (i, 0))],
        out_shape=[jax.ShapeDtypeStruct((N, N), jnp.float32),
                   jax.ShapeDtypeStruct((N, IDXW), jnp.float32),
                   jax.ShapeDtypeStruct((N, IDXW), jnp.int32),
                   jax.ShapeDtypeStruct((N, IDXW), jnp.int32)],
    )(sn, mn, mn)


def _gather_body(sml, stl, mtl, smv, stv, mtv, posl, posv,
                 smbuf, stopbuf, mtopbuf, posbuf):
    c = lax.axis_index("c")
    s = lax.axis_index("s")
    wid = s * 2 + c
    base = pl.multiple_of(wid * RPW, RPW)
    lanes = lax.iota(jnp.int32, 16)
    for sm_h, st_h, mt_h, pos_h in ((sml, stl, mtl, posl),
                                    (smv, stv, mtv, posv)):
        pltpu.sync_copy(st_h.at[pl.ds(base * 8, RPW * 8)], stopbuf)
        pltpu.sync_copy(mt_h, mtopbuf)

        def group(g, carry):
            r0 = pl.multiple_of(base + g * GR, GR)
            pltpu.sync_copy(sm_h.at[pl.ds(r0 * N, GR * N)], smbuf)
            rloc = g * GR + lanes
            acc = jnp.zeros((16,), jnp.float32)
            for t in range(36):
                i, j = t // 6, t % 6
                ii = jnp.full((16,), i, jnp.int32)
                jj = jnp.full((16,), j, jnp.int32)
                idx = plsc.load_gather(stopbuf, [rloc * 8 + ii])
                col = plsc.load_gather(mtopbuf, [idx * 8 + jj])
                acc = acc + plsc.load_gather(smbuf, [lanes * N + col])
            posbuf[pl.ds(pl.multiple_of(g * GR, GR), GR)] = acc
            return carry

        lax.fori_loop(0, NG, group, 0)
        pltpu.sync_copy(posbuf, pos_h.at[pl.ds(base, RPW)])


def _gather(sml, stl, mtl, smv, stv, mtv):
    mesh = plsc.VectorSubcoreMesh(core_axis_name="c", subcore_axis_name="s")
    fn = pl.kernel(
        _gather_body,
        out_type=[jax.ShapeDtypeStruct((N,), jnp.float32),
                  jax.ShapeDtypeStruct((N,), jnp.float32)],
        mesh=mesh,
        scratch_types=[pltpu.VMEM((GR * N,), jnp.float32),
                       pltpu.VMEM((RPW * 8,), jnp.int32),
                       pltpu.VMEM((N * 8,), jnp.int32),
                       pltpu.VMEM((RPW,), jnp.float32)],
        compiler_params=pltpu.CompilerParams(needs_layout_passes=False),
    )
    return fn(sml.reshape(N * N), stl.reshape(N * 8), mtl.reshape(N * 8),
              smv.reshape(N * N), stv.reshape(N * 8), mtv.reshape(N * 8))


def _loss_body(rsl_ref, rsv_ref, pls_ref, pvs_ref, o_ref):
    t = (jnp.sum(jnp.log(rsl_ref[:, 0:1])) - jnp.sum(jnp.log(pls_ref[...]))
         + jnp.sum(jnp.log(rsv_ref[:, 0:1])) - jnp.sum(jnp.log(pvs_ref[...])))
    o_ref[...] = jnp.broadcast_to(t / N, (1, 1))


def _loss(rsl, rsv, posl, posv):
    return pl.pallas_call(
        _loss_body,
        out_shape=jax.ShapeDtypeStruct((1, 1), jnp.float32),
    )(rsl, rsv, posl, posv)


def kernel(stru_feats, visu_feats, ling_feats, Wl1, bl1, Wl2, bl2, gl, betal,
           Wv1, bv1, Wv2, bv2, gv, betav):
    sn = _norm(stru_feats)
    ling, mnl = _fusion(ling_feats, Wl1, bl1, Wl2, bl2, gl, betal)
    visu, mnv = _fusion(visu_feats, Wv1, bv1, Wv2, bv2, gv, betav)
    sml, rsl, stl, mtl = _sim(sn, mnl)
    smv, rsv, stv, mtv = _sim(sn, mnv)
    posl, posv = _gather(sml, stl[:, :8], mtl[:, :8],
                         smv, stv[:, :8], mtv[:, :8])
    loss = _loss(rsl, rsv, posl.reshape(NW, RPW), posv.reshape(NW, RPW))
    return (loss.reshape(()), ling, visu)


# speedup vs baseline: 49.6728x; 1.5017x over previous
import functools

import jax
import jax.numpy as jnp
from jax import lax
from jax.experimental import pallas as pl
from jax.experimental.pallas import tpu as pltpu
from jax.experimental.pallas import tpu_sc as plsc

N = 4096
D = 256
INV_TAU = 2.0
K = 6
BR = 256
NB = N // BR
NW = 32
RPW = N // NW
GR = 16
NG = RPW // GR
_PREC = lax.Precision.HIGHEST


def _norm_body(x_ref, o_ref):
    x = x_ref[...]
    nrm = jnp.sqrt(jnp.sum(x * x, axis=1, keepdims=True))
    o_ref[...] = x / (nrm + 1e-12)


def _norm(x):
    return pl.pallas_call(
        _norm_body,
        out_shape=jax.ShapeDtypeStruct((N, D), jnp.float32),
    )(x)


def _fusion_body(x_ref, w1_ref, b1_ref, w2_ref, b2_ref, g_ref, beta_ref,
                 out_ref, nout_ref):
    h = jnp.dot(x_ref[...], w1_ref[...], precision=_PREC,
                preferred_element_type=jnp.float32) + b1_ref[...]
    h = jnp.where(h >= 0, h, 0.01 * h)
    h = jnp.dot(h, w2_ref[...], precision=_PREC,
                preferred_element_type=jnp.float32) + b2_ref[...]
    mean = jnp.mean(h, axis=0, keepdims=True)
    ctr = h - mean
    var = jnp.mean(ctr * ctr, axis=0, keepdims=True)
    h = ctr / jnp.sqrt(var + 1e-5) * g_ref[...] + beta_ref[...]
    out_ref[...] = h
    nrm = jnp.sqrt(jnp.sum(h * h, axis=1, keepdims=True))
    nout_ref[...] = h / (nrm + 1e-12)


def _fusion(x, w1, b1, w2, b2, g, beta):
    return pl.pallas_call(
        _fusion_body,
        out_shape=[jax.ShapeDtypeStruct((N, D), jnp.float32),
                   jax.ShapeDtypeStruct((N, D), jnp.float32)],
    )(x, w1, b1.reshape(1, D), w2, b2.reshape(1, D),
      g.reshape(1, D), beta.reshape(1, D))


def _topk_cols8(p):
    b = lax.bitcast_convert_type(p, jnp.int32)
    b = jnp.where(b < 0, b ^ jnp.int32(0x7FFFFFFF), b)
    colio = lax.broadcasted_iota(jnp.int32, p.shape, 1)
    cur = (b & jnp.int32(-4096)) | ((N - 1) - colio)
    cols = []
    for _ in range(K):
        m = jnp.max(cur, axis=1, keepdims=True)
        cols.append((N - 1) - (m & jnp.int32(N - 1)))
        cur = jnp.where(cur == m, jnp.int32(-2147483648), cur)
    cols.append(jnp.zeros((p.shape[0], 8 - K), jnp.int32))
    return jnp.concatenate(cols, axis=1)


def _sim_body(sn_ref, mnb_ref, mn_ref, sm_ref, rs_ref, stop_ref, mtop_ref):
    mn16 = mn_ref[...].astype(jnp.bfloat16)
    p = lax.dot_general(sn_ref[...].astype(jnp.bfloat16), mn16,
                        (((1,), (1,)), ((), ())),
                        preferred_element_type=jnp.float32)
    smb = jnp.exp(p * INV_TAU)
    sm_ref[...] = smb
    rs_ref[...] = jnp.broadcast_to(
        jnp.sum(smb, axis=1, keepdims=True), (BR, 8)).reshape(1, BR, 8)
    stop_ref[...] = _topk_cols8(p).reshape(1, BR, 8)
    q = lax.dot_general(mnb_ref[...].astype(jnp.bfloat16), mn16,
                        (((1,), (1,)), ((), ())),
                        preferred_element_type=jnp.float32)
    mtop_ref[...] = _topk_cols8(q).reshape(1, BR, 8)


def _sim(sn, mn):
    sm, rs, stop, mtop = pl.pallas_call(
        _sim_body,
        grid=(NB,),
        in_specs=[pl.BlockSpec((BR, D), lambda i: (i, 0)),
                  pl.BlockSpec((BR, D), lambda i: (i, 0)),
                  pl.BlockSpec((N, D), lambda i: (0, 0))],
        out_specs=[pl.BlockSpec((BR, N), lambda i: (i, 0)),
                   pl.BlockSpec((1, BR, 8), lambda i: (i, 0, 0)),
                   pl.BlockSpec((1, BR, 8), lambda i: (i, 0, 0)),
                   pl.BlockSpec((1, BR, 8), lambda i: (i, 0, 0))],
        out_shape=[jax.ShapeDtypeStruct((N, N), jnp.float32),
                   jax.ShapeDtypeStruct((NB, BR, 8), jnp.float32),
                   jax.ShapeDtypeStruct((NB, BR, 8), jnp.int32),
                   jax.ShapeDtypeStruct((NB, BR, 8), jnp.int32)],
    )(sn, mn, mn)
    return sm, rs.reshape(N, 8), stop.reshape(N, 8), mtop.reshape(N, 8)


def _gather_body(sml, stl, mtl, smv, stv, mtv, posl, posv,
                 smbuf, stopbuf, mtopbuf, posbuf):
    c = lax.axis_index("c")
    s = lax.axis_index("s")
    wid = s * 2 + c
    base = pl.multiple_of(wid * RPW, RPW)
    lanes = lax.iota(jnp.int32, 16)
    for sm_h, st_h, mt_h, pos_h in ((sml, stl, mtl, posl),
                                    (smv, stv, mtv, posv)):
        pltpu.sync_copy(st_h.at[pl.ds(base * 8, RPW * 8)], stopbuf)
        pltpu.sync_copy(mt_h, mtopbuf)

        def group(g, carry):
            r0 = pl.multiple_of(base + g * GR, GR)
            pltpu.sync_copy(sm_h.at[pl.ds(r0 * N, GR * N)], smbuf)
            rloc = g * GR + lanes
            acc = jnp.zeros((16,), jnp.float32)
            for t in range(36):
                i, j = t // 6, t % 6
                ii = jnp.full((16,), i, jnp.int32)
                jj = jnp.full((16,), j, jnp.int32)
                idx = plsc.load_gather(stopbuf, [rloc * 8 + ii])
                col = plsc.load_gather(mtopbuf, [idx * 8 + jj])
                acc = acc + plsc.load_gather(smbuf, [lanes * N + col])
            posbuf[pl.ds(pl.multiple_of(g * GR, GR), GR)] = acc
            return carry

        lax.fori_loop(0, NG, group, 0)
        pltpu.sync_copy(posbuf, pos_h.at[pl.ds(base, RPW)])


def _gather(sml, stl, mtl, smv, stv, mtv):
    mesh = plsc.VectorSubcoreMesh(core_axis_name="c", subcore_axis_name="s")
    fn = pl.kernel(
        _gather_body,
        out_type=[jax.ShapeDtypeStruct((N,), jnp.float32),
                  jax.ShapeDtypeStruct((N,), jnp.float32)],
        mesh=mesh,
        scratch_types=[pltpu.VMEM((GR * N,), jnp.float32),
                       pltpu.VMEM((RPW * 8,), jnp.int32),
                       pltpu.VMEM((N * 8,), jnp.int32),
                       pltpu.VMEM((RPW,), jnp.float32)],
        compiler_params=pltpu.CompilerParams(needs_layout_passes=False),
    )
    return fn(sml.reshape(N * N), stl.reshape(N * 8), mtl.reshape(N * 8),
              smv.reshape(N * N), stv.reshape(N * 8), mtv.reshape(N * 8))


def _loss_body(rsl_ref, rsv_ref, pls_ref, pvs_ref, o_ref):
    t = (jnp.sum(jnp.log(rsl_ref[:, 0:1])) - jnp.sum(jnp.log(pls_ref[...]))
         + jnp.sum(jnp.log(rsv_ref[:, 0:1])) - jnp.sum(jnp.log(pvs_ref[...])))
    o_ref[...] = jnp.broadcast_to(t / N, (1, 1))


def _loss(rsl, rsv, posl, posv):
    return pl.pallas_call(
        _loss_body,
        out_shape=jax.ShapeDtypeStruct((1, 1), jnp.float32),
    )(rsl, rsv, posl, posv)


def kernel(stru_feats, visu_feats, ling_feats, Wl1, bl1, Wl2, bl2, gl, betal,
           Wv1, bv1, Wv2, bv2, gv, betav):
    sn = _norm(stru_feats)
    ling, mnl = _fusion(ling_feats, Wl1, bl1, Wl2, bl2, gl, betal)
    visu, mnv = _fusion(visu_feats, Wv1, bv1, Wv2, bv2, gv, betav)
    sml, rsl, stl, mtl = _sim(sn, mnl)
    smv, rsv, stv, mtv = _sim(sn, mnv)
    posl, posv = _gather(sml, stl, mtl, smv, stv, mtv)
    loss = _loss(rsl, rsv, posl.reshape(NW, RPW), posv.reshape(NW, RPW))
    return (loss.reshape(()), ling, visu)


# speedup vs baseline: 55.3903x; 1.1151x over previous
import functools

import jax
import jax.numpy as jnp
from jax import lax
from jax.experimental import pallas as pl
from jax.experimental.pallas import tpu as pltpu
from jax.experimental.pallas import tpu_sc as plsc

N = 4096
D = 256
INV_TAU = 2.0
K = 6
BR = 256
NB = N // BR
NW = 32
RPW = N // NW
GR = 16
NG = RPW // GR
_PREC = lax.Precision.HIGHEST


def _norm_body(x_ref, o_ref):
    x = x_ref[...]
    nrm = jnp.sqrt(jnp.sum(x * x, axis=1, keepdims=True))
    o_ref[...] = x / (nrm + 1e-12)


def _norm(x):
    return pl.pallas_call(
        _norm_body,
        out_shape=jax.ShapeDtypeStruct((N, D), jnp.float32),
    )(x)


def _fusion_body(x_ref, w1_ref, b1_ref, w2_ref, b2_ref, g_ref, beta_ref,
                 out_ref, nout_ref):
    h = jnp.dot(x_ref[...], w1_ref[...], precision=_PREC,
                preferred_element_type=jnp.float32) + b1_ref[...]
    h = jnp.where(h >= 0, h, 0.01 * h)
    h = jnp.dot(h, w2_ref[...], precision=_PREC,
                preferred_element_type=jnp.float32) + b2_ref[...]
    mean = jnp.mean(h, axis=0, keepdims=True)
    ctr = h - mean
    var = jnp.mean(ctr * ctr, axis=0, keepdims=True)
    h = ctr / jnp.sqrt(var + 1e-5) * g_ref[...] + beta_ref[...]
    out_ref[...] = h
    nrm = jnp.sqrt(jnp.sum(h * h, axis=1, keepdims=True))
    nout_ref[...] = h / (nrm + 1e-12)


def _fusion(x, w1, b1, w2, b2, g, beta):
    return pl.pallas_call(
        _fusion_body,
        out_shape=[jax.ShapeDtypeStruct((N, D), jnp.float32),
                   jax.ShapeDtypeStruct((N, D), jnp.float32)],
    )(x, w1, b1.reshape(1, D), w2, b2.reshape(1, D),
      g.reshape(1, D), beta.reshape(1, D))


def _topk_cols8(p):
    b = lax.bitcast_convert_type(p, jnp.int32)
    b = jnp.where(b < 0, b ^ jnp.int32(0x7FFFFFFF), b)
    colio = lax.broadcasted_iota(jnp.int32, p.shape, 1)
    cur = (b & jnp.int32(-4096)) | ((N - 1) - colio)
    cols = []
    for _ in range(K):
        m = jnp.max(cur, axis=1, keepdims=True)
        cols.append((N - 1) - (m & jnp.int32(N - 1)))
        cur = jnp.where(cur == m, jnp.int32(-2147483648), cur)
    cols.append(jnp.zeros((p.shape[0], 8 - K), jnp.int32))
    return jnp.concatenate(cols, axis=1)


def _sim_body(sn_ref, mnb_ref, mn_ref, sm_ref, rs_ref, stop_ref, mtop_ref):
    mn16 = mn_ref[...].astype(jnp.bfloat16)
    p = lax.dot_general(sn_ref[...].astype(jnp.bfloat16), mn16,
                        (((1,), (1,)), ((), ())),
                        preferred_element_type=jnp.float32)
    smb = jnp.exp(p * INV_TAU)
    sm_ref[...] = smb
    rs_ref[...] = jnp.broadcast_to(
        jnp.sum(smb, axis=1, keepdims=True), (BR, 8)).reshape(1, BR, 8)
    stop_ref[...] = _topk_cols8(p).reshape(1, BR, 8)
    q = lax.dot_general(mnb_ref[...].astype(jnp.bfloat16), mn16,
                        (((1,), (1,)), ((), ())),
                        preferred_element_type=jnp.float32)
    mtop_ref[...] = _topk_cols8(q).reshape(1, BR, 8)


def _sim(sn, mn):
    sm, rs, stop, mtop = pl.pallas_call(
        _sim_body,
        grid=(NB,),
        in_specs=[pl.BlockSpec((BR, D), lambda i: (i, 0)),
                  pl.BlockSpec((BR, D), lambda i: (i, 0)),
                  pl.BlockSpec((N, D), lambda i: (0, 0))],
        out_specs=[pl.BlockSpec((BR, N), lambda i: (i, 0)),
                   pl.BlockSpec((1, BR, 8), lambda i: (i, 0, 0)),
                   pl.BlockSpec((1, BR, 8), lambda i: (i, 0, 0)),
                   pl.BlockSpec((1, BR, 8), lambda i: (i, 0, 0))],
        out_shape=[jax.ShapeDtypeStruct((N, N), jnp.float32),
                   jax.ShapeDtypeStruct((NB, BR, 8), jnp.float32),
                   jax.ShapeDtypeStruct((NB, BR, 8), jnp.int32),
                   jax.ShapeDtypeStruct((NB, BR, 8), jnp.int32)],
    )(sn, mn, mn)
    return sm, rs.reshape(N, 8), stop.reshape(N, 8), mtop.reshape(N, 8)


def _gather_body(sml, stl, mtl, smv, stv, mtv, posl, posv,
                 smbuf, stopbuf, mtopbuf, posbuf):
    c = lax.axis_index("c")
    s = lax.axis_index("s")
    wid = s * 2 + c
    base = pl.multiple_of(wid * RPW, RPW)
    lanes = lax.iota(jnp.int32, 16)
    for sm_h, st_h, mt_h, pos_h in ((sml, stl, mtl, posl),
                                    (smv, stv, mtv, posv)):
        pltpu.sync_copy(st_h.at[pl.ds(base * 8, RPW * 8)], stopbuf)
        pltpu.sync_copy(mt_h, mtopbuf)

        def group(g, carry):
            r0 = pl.multiple_of(base + g * GR, GR)
            pltpu.sync_copy(sm_h.at[pl.ds(r0, GR), :], smbuf)
            rloc = g * GR + lanes
            acc = jnp.zeros((16,), jnp.float32)
            for t in range(36):
                i, j = t // 6, t % 6
                ii = jnp.full((16,), i, jnp.int32)
                jj = jnp.full((16,), j, jnp.int32)
                idx = plsc.load_gather(stopbuf, [rloc * 8 + ii])
                col = plsc.load_gather(mtopbuf, [idx * 8 + jj])
                acc = acc + plsc.load_gather(smbuf, [lanes, col])
            posbuf[pl.ds(pl.multiple_of(g * GR, GR), GR)] = acc
            return carry

        lax.fori_loop(0, NG, group, 0)
        pltpu.sync_copy(posbuf, pos_h.at[pl.ds(base, RPW)])


def _gather(sml, stl, mtl, smv, stv, mtv):
    mesh = plsc.VectorSubcoreMesh(core_axis_name="c", subcore_axis_name="s")
    fn = pl.kernel(
        _gather_body,
        out_type=[jax.ShapeDtypeStruct((N,), jnp.float32),
                  jax.ShapeDtypeStruct((N,), jnp.float32)],
        mesh=mesh,
        scratch_types=[pltpu.VMEM((GR, N), jnp.float32),
                       pltpu.VMEM((RPW * 8,), jnp.int32),
                       pltpu.VMEM((N * 8,), jnp.int32),
                       pltpu.VMEM((RPW,), jnp.float32)],
        compiler_params=pltpu.CompilerParams(needs_layout_passes=False),
    )
    return fn(sml, stl.reshape(N * 8), mtl.reshape(N * 8),
              smv, stv.reshape(N * 8), mtv.reshape(N * 8))


def _loss_body(rsl_ref, rsv_ref, pls_ref, pvs_ref, o_ref):
    t = (jnp.sum(jnp.log(rsl_ref[:, 0:1])) - jnp.sum(jnp.log(pls_ref[...]))
         + jnp.sum(jnp.log(rsv_ref[:, 0:1])) - jnp.sum(jnp.log(pvs_ref[...])))
    o_ref[...] = jnp.broadcast_to(t / N, (1, 1))


def _loss(rsl, rsv, posl, posv):
    return pl.pallas_call(
        _loss_body,
        out_shape=jax.ShapeDtypeStruct((1, 1), jnp.float32),
    )(rsl, rsv, posl, posv)


def kernel(stru_feats, visu_feats, ling_feats, Wl1, bl1, Wl2, bl2, gl, betal,
           Wv1, bv1, Wv2, bv2, gv, betav):
    sn = _norm(stru_feats)
    ling, mnl = _fusion(ling_feats, Wl1, bl1, Wl2, bl2, gl, betal)
    visu, mnv = _fusion(visu_feats, Wv1, bv1, Wv2, bv2, gv, betav)
    sml, rsl, stl, mtl = _sim(sn, mnl)
    smv, rsv, stv, mtv = _sim(sn, mnv)
    posl, posv = _gather(sml, stl, mtl, smv, stv, mtv)
    loss = _loss(rsl, rsv, posl.reshape(NW, RPW), posv.reshape(NW, RPW))
    return (loss.reshape(()), ling, visu)


# speedup vs baseline: 68.1856x; 1.2310x over previous
import functools

import jax
import jax.numpy as jnp
from jax import lax
from jax.experimental import pallas as pl
from jax.experimental.pallas import tpu as pltpu
from jax.experimental.pallas import tpu_sc as plsc

N = 4096
D = 256
INV_TAU = 2.0
K = 6
BR = 256
NB = N // BR
NW = 32
RPW = N // NW
GR = 16
NG = RPW // GR


def _norm_body(x_ref, o_ref):
    x = x_ref[...]
    nrm = jnp.sqrt(jnp.sum(x * x, axis=1, keepdims=True))
    o_ref[...] = x / (nrm + 1e-12)


def _norm(x):
    return pl.pallas_call(
        _norm_body,
        out_shape=jax.ShapeDtypeStruct((N, D), jnp.float32),
    )(x)


def _fusion_body(x_ref, w1_ref, b1_ref, w2_ref, b2_ref, g_ref, beta_ref,
                 out_ref, nout_ref):
    h = jnp.dot(x_ref[...], w1_ref[...],
                preferred_element_type=jnp.float32) + b1_ref[...]
    h = jnp.where(h >= 0, h, 0.01 * h)
    h = jnp.dot(h, w2_ref[...],
                preferred_element_type=jnp.float32) + b2_ref[...]
    mean = jnp.mean(h, axis=0, keepdims=True)
    ctr = h - mean
    var = jnp.mean(ctr * ctr, axis=0, keepdims=True)
    h = ctr / jnp.sqrt(var + 1e-5) * g_ref[...] + beta_ref[...]
    out_ref[...] = h
    nrm = jnp.sqrt(jnp.sum(h * h, axis=1, keepdims=True))
    nout_ref[...] = h / (nrm + 1e-12)


def _fusion(x, w1, b1, w2, b2, g, beta):
    return pl.pallas_call(
        _fusion_body,
        out_shape=[jax.ShapeDtypeStruct((N, D), jnp.float32),
                   jax.ShapeDtypeStruct((N, D), jnp.float32)],
    )(x, w1, b1.reshape(1, D), w2, b2.reshape(1, D),
      g.reshape(1, D), beta.reshape(1, D))


def _topk_cols8(p):
    b = lax.bitcast_convert_type(p, jnp.int32)
    bu = jnp.where(b < 0, ~b, b | jnp.int32(-2147483648))
    colio = lax.broadcasted_iota(jnp.int32, p.shape, 1)
    ki = (lax.shift_right_logical(bu, 1) & jnp.int32(~4095)) | ((N - 1) - colio)
    kf = lax.bitcast_convert_type(ki, jnp.float32)
    cols = []
    m = None
    for t in range(K):
        if t == 0:
            m = jnp.max(kf, axis=1, keepdims=True)
        else:
            m = jnp.max(jnp.where(kf < m, kf, jnp.float32(-1.0)),
                        axis=1, keepdims=True)
        mi = lax.bitcast_convert_type(m, jnp.int32)
        cols.append((N - 1) - (mi & jnp.int32(N - 1)))
    cols.append(jnp.zeros((p.shape[0], 8 - K), jnp.int32))
    return jnp.concatenate(cols, axis=1)


def _sim_body(sn_ref, mnb_ref, mn_ref, sm_ref, rs_ref, stop_ref, mtop_ref):
    mn16 = mn_ref[...].astype(jnp.bfloat16)
    p = lax.dot_general(sn_ref[...].astype(jnp.bfloat16), mn16,
                        (((1,), (1,)), ((), ())),
                        preferred_element_type=jnp.float32)
    smb = jnp.exp(p * INV_TAU)
    sm_ref[...] = smb
    rs_ref[...] = jnp.broadcast_to(
        jnp.sum(smb, axis=1, keepdims=True), (BR, 8)).reshape(1, BR, 8)
    stop_ref[...] = _topk_cols8(p).reshape(1, BR, 8)
    q = lax.dot_general(mnb_ref[...].astype(jnp.bfloat16), mn16,
                        (((1,), (1,)), ((), ())),
                        preferred_element_type=jnp.float32)
    mtop_ref[...] = _topk_cols8(q).reshape(1, BR, 8)


def _sim(sn, mn):
    sm, rs, stop, mtop = pl.pallas_call(
        _sim_body,
        grid=(NB,),
        in_specs=[pl.BlockSpec((BR, D), lambda i: (i, 0)),
                  pl.BlockSpec((BR, D), lambda i: (i, 0)),
                  pl.BlockSpec((N, D), lambda i: (0, 0))],
        out_specs=[pl.BlockSpec((BR, N), lambda i: (i, 0)),
                   pl.BlockSpec((1, BR, 8), lambda i: (i, 0, 0)),
                   pl.BlockSpec((1, BR, 8), lambda i: (i, 0, 0)),
                   pl.BlockSpec((1, BR, 8), lambda i: (i, 0, 0))],
        out_shape=[jax.ShapeDtypeStruct((N, N), jnp.float32),
                   jax.ShapeDtypeStruct((NB, BR, 8), jnp.float32),
                   jax.ShapeDtypeStruct((NB, BR, 8), jnp.int32),
                   jax.ShapeDtypeStruct((NB, BR, 8), jnp.int32)],
    )(sn, mn, mn)
    return sm, rs.reshape(N, 8), stop.reshape(N, 8), mtop.reshape(N, 8)


def _gather_body(sml, stl, mtl, smv, stv, mtv, posl, posv,
                 smbuf, stopbuf, mtopbuf, posbuf):
    c = lax.axis_index("c")
    s = lax.axis_index("s")
    wid = s * 2 + c
    base = pl.multiple_of(wid * RPW, RPW)
    lanes = lax.iota(jnp.int32, 16)
    for sm_h, st_h, mt_h, pos_h in ((sml, stl, mtl, posl),
                                    (smv, stv, mtv, posv)):
        pltpu.sync_copy(st_h.at[pl.ds(base * 8, RPW * 8)], stopbuf)
        pltpu.sync_copy(mt_h, mtopbuf)

        def group(g, carry):
            r0 = pl.multiple_of(base + g * GR, GR)
            pltpu.sync_copy(sm_h.at[pl.ds(r0, GR), :], smbuf)
            rloc = g * GR + lanes
            acc = jnp.zeros((16,), jnp.float32)
            for t in range(36):
                i, j = t // 6, t % 6
                ii = jnp.full((16,), i, jnp.int32)
                jj = jnp.full((16,), j, jnp.int32)
                idx = plsc.load_gather(stopbuf, [rloc * 8 + ii])
                col = plsc.load_gather(mtopbuf, [idx * 8 + jj])
                acc = acc + plsc.load_gather(smbuf, [lanes, col])
            posbuf[pl.ds(pl.multiple_of(g * GR, GR), GR)] = acc
            return carry

        lax.fori_loop(0, NG, group, 0)
        pltpu.sync_copy(posbuf, pos_h.at[pl.ds(base, RPW)])


def _gather(sml, stl, mtl, smv, stv, mtv):
    mesh = plsc.VectorSubcoreMesh(core_axis_name="c", subcore_axis_name="s")
    fn = pl.kernel(
        _gather_body,
        out_type=[jax.ShapeDtypeStruct((N,), jnp.float32),
                  jax.ShapeDtypeStruct((N,), jnp.float32)],
        mesh=mesh,
        scratch_types=[pltpu.VMEM((GR, N), jnp.float32),
                       pltpu.VMEM((RPW * 8,), jnp.int32),
                       pltpu.VMEM((N * 8,), jnp.int32),
                       pltpu.VMEM((RPW,), jnp.float32)],
        compiler_params=pltpu.CompilerParams(needs_layout_passes=False),
    )
    return fn(sml, stl.reshape(N * 8), mtl.reshape(N * 8),
              smv, stv.reshape(N * 8), mtv.reshape(N * 8))


def _loss_body(rsl_ref, rsv_ref, pls_ref, pvs_ref, o_ref):
    t = (jnp.sum(jnp.log(rsl_ref[:, 0:1])) - jnp.sum(jnp.log(pls_ref[...]))
         + jnp.sum(jnp.log(rsv_ref[:, 0:1])) - jnp.sum(jnp.log(pvs_ref[...])))
    o_ref[...] = jnp.broadcast_to(t / N, (1, 1))


def _loss(rsl, rsv, posl, posv):
    return pl.pallas_call(
        _loss_body,
        out_shape=jax.ShapeDtypeStruct((1, 1), jnp.float32),
    )(rsl, rsv, posl, posv)


def kernel(stru_feats, visu_feats, ling_feats, Wl1, bl1, Wl2, bl2, gl, betal,
           Wv1, bv1, Wv2, bv2, gv, betav):
    sn = _norm(stru_feats)
    ling, mnl = _fusion(ling_feats, Wl1, bl1, Wl2, bl2, gl, betal)
    visu, mnv = _fusion(visu_feats, Wv1, bv1, Wv2, bv2, gv, betav)
    sml, rsl, stl, mtl = _sim(sn, mnl)
    smv, rsv, stv, mtv = _sim(sn, mnv)
    posl, posv = _gather(sml, stl, mtl, smv, stv, mtv)
    loss = _loss(rsl, rsv, posl.reshape(NW, RPW), posv.reshape(NW, RPW))
    return (loss.reshape(()), ling, visu)


# speedup vs baseline: 72.5011x; 1.0633x over previous
import functools

import jax
import jax.numpy as jnp
from jax import lax
from jax.experimental import pallas as pl
from jax.experimental.pallas import tpu as pltpu
from jax.experimental.pallas import tpu_sc as plsc

N = 4096
D = 256
INV_TAU = 2.0
K = 6
BR = 256
NB = N // BR
NW = 32
RPW = N // NW
GR = 16
NG = RPW // GR


def _fuse_into(x_ref, w1_ref, b1_ref, w2_ref, b2_ref, g_ref, beta_ref,
               out_ref, nout_ref):
    h = jnp.dot(x_ref[...], w1_ref[...],
                preferred_element_type=jnp.float32) + b1_ref[...]
    h = jnp.where(h >= 0, h, 0.01 * h)
    h = jnp.dot(h, w2_ref[...],
                preferred_element_type=jnp.float32) + b2_ref[...]
    mean = jnp.mean(h, axis=0, keepdims=True)
    ctr = h - mean
    var = jnp.mean(ctr * ctr, axis=0, keepdims=True)
    h = ctr / jnp.sqrt(var + 1e-5) * g_ref[...] + beta_ref[...]
    out_ref[...] = h
    nrm = jnp.sqrt(jnp.sum(h * h, axis=1, keepdims=True))
    nout_ref[...] = h / (nrm + 1e-12)


def _prep_body(st_ref, xl_ref, wl1, bl1, wl2, bl2, gl, betal,
               xv_ref, wv1, bv1, wv2, bv2, gv, betav,
               sn_ref, lo_ref, ln_ref, vo_ref, vn_ref):
    x = st_ref[...]
    nrm = jnp.sqrt(jnp.sum(x * x, axis=1, keepdims=True))
    sn_ref[...] = x / (nrm + 1e-12)
    _fuse_into(xl_ref, wl1, bl1, wl2, bl2, gl, betal, lo_ref, ln_ref)
    _fuse_into(xv_ref, wv1, bv1, wv2, bv2, gv, betav, vo_ref, vn_ref)


def _prep(stru, xl, wl1, bl1, wl2, bl2, gl, betal,
          xv, wv1, bv1, wv2, bv2, gv, betav):
    return pl.pallas_call(
        _prep_body,
        out_shape=[jax.ShapeDtypeStruct((N, D), jnp.float32)] * 5,
    )(stru, xl, wl1, bl1.reshape(1, D), wl2, bl2.reshape(1, D),
      gl.reshape(1, D), betal.reshape(1, D),
      xv, wv1, bv1.reshape(1, D), wv2, bv2.reshape(1, D),
      gv.reshape(1, D), betav.reshape(1, D))


def _topk_cols8(p):
    b = lax.bitcast_convert_type(p, jnp.int32)
    bu = jnp.where(b < 0, ~b, b | jnp.int32(-2147483648))
    colio = lax.broadcasted_iota(jnp.int32, p.shape, 1)
    ki = (lax.shift_right_logical(bu, 1) & jnp.int32(~4095)) | ((N - 1) - colio)
    kf = lax.bitcast_convert_type(ki, jnp.float32)
    cols = []
    m = jnp.full((p.shape[0], 1), 1e30, jnp.float32)
    for _ in range(K):
        m = jnp.max(jnp.where(kf < m, kf, jnp.float32(-1.0)),
                    axis=1, keepdims=True)
        mi = lax.bitcast_convert_type(m, jnp.int32)
        cols.append((N - 1) - (mi & jnp.int32(N - 1)))
    cols.append(jnp.zeros((p.shape[0], 8 - K), jnp.int32))
    return jnp.concatenate(cols, axis=1)


def _sim_body(sn_ref, mnb_ref, mn_ref, sm_ref, rs_ref, stop_ref, mtop_ref):
    mn16 = mn_ref[...].astype(jnp.bfloat16)
    p = lax.dot_general(sn_ref[...].astype(jnp.bfloat16), mn16,
                        (((1,), (1,)), ((), ())),
                        preferred_element_type=jnp.float32)
    smb = jnp.exp(p * INV_TAU)
    sm_ref[...] = pltpu.bitcast(smb.astype(jnp.bfloat16), jnp.int32)
    rs_ref[...] = jnp.broadcast_to(
        jnp.sum(smb, axis=1, keepdims=True), (BR, 8)).reshape(1, BR, 8)
    stop_ref[...] = _topk_cols8(p).reshape(1, BR, 8)
    q = lax.dot_general(mnb_ref[...].astype(jnp.bfloat16), mn16,
                        (((1,), (1,)), ((), ())),
                        preferred_element_type=jnp.float32)
    mtop_ref[...] = _topk_cols8(q).reshape(1, BR, 8)


def _sim(sn, mn):
    sm, rs, stop, mtop = pl.pallas_call(
        _sim_body,
        grid=(NB,),
        in_specs=[pl.BlockSpec((BR, D), lambda i: (i, 0)),
                  pl.BlockSpec((BR, D), lambda i: (i, 0)),
                  pl.BlockSpec((N, D), lambda i: (0, 0))],
        out_specs=[pl.BlockSpec((BR // 2, N), lambda i: (i, 0)),
                   pl.BlockSpec((1, BR, 8), lambda i: (i, 0, 0)),
                   pl.BlockSpec((1, BR, 8), lambda i: (i, 0, 0)),
                   pl.BlockSpec((1, BR, 8), lambda i: (i, 0, 0))],
        out_shape=[jax.ShapeDtypeStruct((N // 2, N), jnp.int32),
                   jax.ShapeDtypeStruct((NB, BR, 8), jnp.float32),
                   jax.ShapeDtypeStruct((NB, BR, 8), jnp.int32),
                   jax.ShapeDtypeStruct((NB, BR, 8), jnp.int32)],
    )(sn, mn, mn)
    return sm, rs.reshape(N, 8), stop.reshape(N, 8), mtop.reshape(N, 8)


def _gather_body(sm_h, st_h, mt_h, pos_h, smbuf, stopbuf, mtopbuf, posbuf):
    c = lax.axis_index("c")
    s = lax.axis_index("s")
    wid = s * 2 + c
    base = pl.multiple_of(wid * RPW, RPW)
    lanes = lax.iota(jnp.int32, 16)
    pltpu.sync_copy(st_h.at[pl.ds(base * 8, RPW * 8)], stopbuf)
    pltpu.sync_copy(mt_h, mtopbuf)

    halfw = lax.shift_right_logical(lanes, 1)
    odd = (lanes & 1) == 1

    def group(g, carry):
        r0 = pl.multiple_of((base + g * GR) // 2, GR // 2)
        pltpu.sync_copy(sm_h.at[pl.ds(r0, GR // 2), :], smbuf)
        rloc = g * GR + lanes
        acc = jnp.zeros((16,), jnp.float32)
        for t in range(36):
            i, j = t // 6, t % 6
            ii = jnp.full((16,), i, jnp.int32)
            jj = jnp.full((16,), j, jnp.int32)
            idx = plsc.load_gather(stopbuf, [rloc * 8 + ii])
            col = plsc.load_gather(mtopbuf, [idx * 8 + jj])
            w = plsc.load_gather(smbuf, [halfw, col])
            bits = jnp.where(odd, w & jnp.int32(-65536),
                             lax.shift_left(w, 16))
            acc = acc + plsc.bitcast(bits, jnp.float32)
        posbuf[pl.ds(pl.multiple_of(g * GR, GR), GR)] = acc
        return carry

    lax.fori_loop(0, NG, group, 0)
    pltpu.sync_copy(posbuf, pos_h.at[pl.ds(base, RPW)])


def _gather(sm, st, mt):
    mesh = plsc.VectorSubcoreMesh(core_axis_name="c", subcore_axis_name="s")
    fn = pl.kernel(
        _gather_body,
        out_type=jax.ShapeDtypeStruct((N,), jnp.float32),
        mesh=mesh,
        scratch_types=[pltpu.VMEM((GR // 2, N), jnp.int32),
                       pltpu.VMEM((RPW * 8,), jnp.int32),
                       pltpu.VMEM((N * 8,), jnp.int32),
                       pltpu.VMEM((RPW,), jnp.float32)],
        compiler_params=pltpu.CompilerParams(needs_layout_passes=False),
    )
    return fn(sm, st.reshape(N * 8), mt.reshape(N * 8))


def _loss_body(rsl_ref, rsv_ref, pls_ref, pvs_ref, o_ref):
    t = (jnp.sum(jnp.log(rsl_ref[:, 0:1])) - jnp.sum(jnp.log(pls_ref[...]))
         + jnp.sum(jnp.log(rsv_ref[:, 0:1])) - jnp.sum(jnp.log(pvs_ref[...])))
    o_ref[...] = jnp.broadcast_to(t / N, (1, 1))


def _loss(rsl, rsv, posl, posv):
    return pl.pallas_call(
        _loss_body,
        out_shape=jax.ShapeDtypeStruct((1, 1), jnp.float32),
    )(rsl, rsv, posl, posv)


def kernel(stru_feats, visu_feats, ling_feats, Wl1, bl1, Wl2, bl2, gl, betal,
           Wv1, bv1, Wv2, bv2, gv, betav):
    sn, ling, mnl, visu, mnv = _prep(
        stru_feats, ling_feats, Wl1, bl1, Wl2, bl2, gl, betal,
        visu_feats, Wv1, bv1, Wv2, bv2, gv, betav)
    sml, rsl, stl, mtl = _sim(sn, mnl)
    posl = _gather(sml, stl, mtl)
    smv, rsv, stv, mtv = _sim(sn, mnv)
    posv = _gather(smv, stv, mtv)
    loss = _loss(rsl, rsv, posl.reshape(NW, RPW), posv.reshape(NW, RPW))
    return (loss.reshape(()), ling, visu)
